# baseline (speedup 1.0000x reference)
import functools

import jax
import jax.numpy as jnp
from jax import lax
from jax.experimental import pallas as pl
from jax.experimental.pallas import tpu as pltpu
from jax.experimental.pallas import tpu_sc as plsc

MAX_TOKENS = 1024
PAD_VALUE = 0.0

_SEQ_BLK = 256


def _fill_body(tok_ref, m_ref, out_ref):
    m = m_ref[0]
    out_ref[0] = jnp.where(m != 0, jnp.float32(PAD_VALUE), tok_ref[0])


def _masked_fill(obs_tokens, mask_col):
    batch, seq_len, dim = obs_tokens.shape
    n_seq = MAX_TOKENS // _SEQ_BLK
    return pl.pallas_call(
        _fill_body,
        grid=(batch, n_seq),
        in_specs=[
            pl.BlockSpec((1, _SEQ_BLK, dim), lambda b, s: (b, s, 0)),
            pl.BlockSpec((1, _SEQ_BLK, 1), lambda b, s: (b, s, 0)),
        ],
        out_specs=pl.BlockSpec((1, _SEQ_BLK, dim), lambda b, s: (b, s, 0)),
        out_shape=jax.ShapeDtypeStruct((batch, MAX_TOKENS, dim), obs_tokens.dtype),
    )(obs_tokens, mask_col)


def _lengths_sc(mask_t):
    n, batch = mask_t.shape
    mask_flat = mask_t.reshape(n * batch)
    mesh = plsc.VectorSubcoreMesh(core_axis_name="c", subcore_axis_name="s")

    @functools.partial(
        pl.kernel,
        mesh=mesh,
        out_type=jax.ShapeDtypeStruct((batch,), jnp.int32),
        scratch_types=[
            pltpu.VMEM((n * batch,), jnp.int32),
            pltpu.VMEM((batch,), jnp.int32),
        ],
    )
    def k(mask_hbm, out_hbm, buf_v, res_v):
        cid = lax.axis_index("c")
        sid = lax.axis_index("s")

        @pl.when(jnp.logical_and(cid == 0, sid == 0))
        def _():
            pltpu.sync_copy(mask_hbm, buf_v)

            def body(i, acc):
                return acc + buf_v[pl.ds(i * batch, batch)]

            acc = lax.fori_loop(0, n, body, jnp.zeros((batch,), jnp.int32))
            res_v[...] = n - acc
            pltpu.sync_copy(res_v, out_hbm)

    return k(mask_flat)


def kernel(obs_tokens, obs_mask):
    new_mask = obs_mask[:, :MAX_TOKENS]
    mask_col = new_mask[:, :, None].astype(jnp.int32)
    trimmed = _masked_fill(obs_tokens, mask_col)
    lengths = _lengths_sc(mask_col[:, :, 0].T)
    return trimmed, new_mask, lengths
